# Initial kernel scaffold; baseline (speedup 1.0000x reference)
import jax
import jax.numpy as jnp
from jax.experimental import pallas as pl


def kernel(T, weight):
    raise NotImplementedError("write your pallas kernel here")


# speedup vs baseline: 1.3741x; 1.3741x over previous
import functools

import jax
import jax.numpy as jnp
from jax import lax
from jax.experimental import pallas as pl
from jax.experimental.pallas import tpu as pltpu
from jax.experimental.pallas import tpu_sc as plsc

R = 8192
D = 768
NC = 2
NS = 16
NW = NC * NS
ROWS_PER_W = R // NW
CH = 64
N_CHUNKS = ROWS_PER_W // CH


@functools.partial(
    pl.kernel,
    out_type=jax.ShapeDtypeStruct((R, D), jnp.float32),
    mesh=plsc.VectorSubcoreMesh(core_axis_name="c", subcore_axis_name="s"),
    scratch_types=[
        pltpu.VMEM((16,), jnp.int32),
        pltpu.VMEM((1, D), jnp.float32),
        pltpu.VMEM((CH, D), jnp.float32),
    ],
)
def _emb_lookup(t_hbm, w_hbm, out_hbm, tvec_v, rowbuf_v, buf_v):
    wid = lax.axis_index("s") * NC + lax.axis_index("c")
    base = wid * ROWS_PER_W

    pltpu.sync_copy(t_hbm, tvec_v)
    t = tvec_v[...][0]
    tl = jnp.clip(t, 1, R)

    for c in range(N_CHUNKS):
        s = base + c * CH
        pltpu.sync_copy(w_hbm.at[pl.ds(s, CH)], buf_v)
        pltpu.sync_copy(buf_v, out_hbm.at[pl.ds(s, CH)])

    pltpu.sync_copy(w_hbm.at[pl.ds(tl - 1, 1)], rowbuf_v)
    start = jnp.maximum(tl, base)

    def _fix(i, carry):
        pltpu.sync_copy(rowbuf_v, out_hbm.at[pl.ds(i, 1)])
        return carry

    lax.fori_loop(start, base + ROWS_PER_W, _fix, 0)


def kernel(T, weight):
    t_arr = jnp.full((16,), jnp.asarray(T, jnp.int32))
    return _emb_lookup(t_arr, weight)


# speedup vs baseline: 1.4398x; 1.0478x over previous
import functools

import jax
import jax.numpy as jnp
from jax import lax
from jax.experimental import pallas as pl
from jax.experimental.pallas import tpu as pltpu
from jax.experimental.pallas import tpu_sc as plsc

R = 8192
D = 768
NC = 2
NS = 16
NW = NC * NS
ROWS_PER_W = R // NW
CH = 64
N_CHUNKS = ROWS_PER_W // CH


@functools.partial(
    pl.kernel,
    out_type=jax.ShapeDtypeStruct((R, D), jnp.float32),
    mesh=plsc.VectorSubcoreMesh(core_axis_name="c", subcore_axis_name="s"),
    scratch_types=[
        pltpu.VMEM((16,), jnp.int32),
        pltpu.VMEM((1, D), jnp.float32),
        pltpu.VMEM((CH, D), jnp.float32),
        pltpu.VMEM((CH, D), jnp.float32),
        pltpu.SemaphoreType.DMA,
        pltpu.SemaphoreType.DMA,
        pltpu.SemaphoreType.DMA,
        pltpu.SemaphoreType.DMA,
    ],
)
def _emb_lookup(t_hbm, w_hbm, out_hbm, tvec_v, rowbuf_v, buf0_v, buf1_v,
                rsem0, rsem1, wsem0, wsem1):
    wid = lax.axis_index("s") * NC + lax.axis_index("c")
    base = wid * ROWS_PER_W
    bufs = (buf0_v, buf1_v)
    rsems = (rsem0, rsem1)
    wsems = (wsem0, wsem1)

    reads = [None] * N_CHUNKS
    writes = [None] * N_CHUNKS
    reads[0] = pltpu.async_copy(w_hbm.at[pl.ds(base, CH)], bufs[0], rsems[0])

    pltpu.sync_copy(t_hbm, tvec_v)
    t = tvec_v[...][0]
    tl = jnp.clip(t, 1, R)

    for c in range(N_CHUNKS):
        b = c % 2
        reads[c].wait()
        if c >= 1:
            writes[c - 1].wait()
        if c + 1 < N_CHUNKS:
            nb = (c + 1) % 2
            reads[c + 1] = pltpu.async_copy(
                w_hbm.at[pl.ds(base + (c + 1) * CH, CH)], bufs[nb], rsems[nb])
        writes[c] = pltpu.async_copy(
            bufs[b], out_hbm.at[pl.ds(base + c * CH, CH)], wsems[b])
    writes[N_CHUNKS - 1].wait()

    pltpu.sync_copy(w_hbm.at[pl.ds(tl - 1, 1)], rowbuf_v)
    start = jnp.maximum(tl, base)

    def _fix(i, carry):
        pltpu.sync_copy(rowbuf_v, out_hbm.at[pl.ds(i, 1)])
        return carry

    lax.fori_loop(start, base + ROWS_PER_W, _fix, 0)


def kernel(T, weight):
    t_arr = jnp.full((16,), jnp.asarray(T, jnp.int32))
    return _emb_lookup(t_arr, weight)
